# baseline (speedup 1.0000x reference)
import jax
import jax.numpy as jnp
from jax.experimental import pallas as pl
from jax.experimental.pallas import tpu as pltpu

_BM = 1024
_BN = 1024
_BS = 128
_N = 4096
_K = 64
_KP = 128


def _prep_body(x_ref, y_ref, a2_ref, b2_ref):
    x = x_ref[0]
    y = y_ref[0]
    xn = jnp.sum(x * x, axis=1, keepdims=True)
    yn = jnp.sum(y * y, axis=1, keepdims=True)
    ones = jnp.ones((_BM, 1), jnp.float32)
    zeros = jnp.zeros((_BM, _KP - _K - 2), jnp.float32)
    a2_ref[0] = jnp.concatenate(
        [x * -2.0, xn, ones, zeros], axis=1).astype(jnp.bfloat16)
    b2_ref[0] = jnp.concatenate(
        [y, ones, yn, zeros], axis=1).astype(jnp.bfloat16)


def _main_body(a2_ref, b2_ref, out_ref, acc_ref, racc_ref, cacc_ref):
    b_b = pl.program_id(0)
    b_i = pl.program_id(1)
    b_j = pl.program_id(2)
    nb = pl.num_programs(0)
    ni = pl.num_programs(1)
    nj = pl.num_programs(2)

    @pl.when((b_b == 0) & (b_i == 0) & (b_j == 0))
    def _init_acc():
        acc_ref[0, 0] = 0.0

    @pl.when(b_j == 0)
    def _init_racc():
        racc_ref[...] = jnp.full((_BM, _BS), jnp.inf, jnp.bfloat16)

    @pl.when((b_i == 0) & (b_j == 0))
    def _init_cacc():
        cacc_ref[...] = jnp.full((8, _N), jnp.inf, jnp.bfloat16)

    a2 = a2_ref[0]
    for s in range(_BN // _BS):
        b2s = b2_ref[0, s * _BS:(s + 1) * _BS, :]
        d = jax.lax.dot_general(
            a2, b2s, (((1,), (1,)), ((), ())),
            preferred_element_type=jnp.float32).astype(jnp.bfloat16)
        racc_ref[...] = jnp.minimum(racc_ref[...], d)
        cp8 = jnp.min(d.reshape(_BM // 8, 8, _BS), axis=0)
        csl = (slice(None), pl.ds(b_j * _BN + s * _BS, _BS))
        cacc_ref[csl] = jnp.minimum(cacc_ref[csl], cp8)

    @pl.when(b_j == nj - 1)
    def _fin_rows():
        acc_ref[0, 0] += jnp.sum(
            jnp.min(racc_ref[...], axis=1).astype(jnp.float32))

    @pl.when(b_i == ni - 1)
    def _fin_cols():
        acc_ref[0, 0] += jnp.sum(
            jnp.min(cacc_ref[:, pl.ds(b_j * _BN, _BN)],
                    axis=0).astype(jnp.float32))

    @pl.when((b_b == nb - 1) & (b_i == ni - 1) & (b_j == nj - 1))
    def _write_out():
        out_ref[...] = jnp.full((1, 1), acc_ref[0, 0], jnp.float32)


def kernel(input, output):
    nb, n, k = input.shape
    a2, b2 = pl.pallas_call(
        _prep_body,
        grid=(nb, n // _BM),
        in_specs=[
            pl.BlockSpec((1, _BM, k), lambda b, i: (b, i, 0)),
            pl.BlockSpec((1, _BM, k), lambda b, i: (b, i, 0)),
        ],
        out_specs=[
            pl.BlockSpec((1, _BM, _KP), lambda b, i: (b, i, 0)),
            pl.BlockSpec((1, _BM, _KP), lambda b, i: (b, i, 0)),
        ],
        out_shape=[
            jax.ShapeDtypeStruct((nb, n, _KP), jnp.bfloat16),
            jax.ShapeDtypeStruct((nb, n, _KP), jnp.bfloat16),
        ],
    )(input, output)

    res = pl.pallas_call(
        _main_body,
        grid=(nb, n // _BM, n // _BN),
        in_specs=[
            pl.BlockSpec((1, _BM, _KP), lambda b, i, j: (b, i, 0)),
            pl.BlockSpec((1, _BN, _KP), lambda b, i, j: (b, j, 0)),
        ],
        out_specs=pl.BlockSpec((1, 1), lambda b, i, j: (0, 0)),
        out_shape=jax.ShapeDtypeStruct((1, 1), jnp.float32),
        scratch_shapes=[
            pltpu.SMEM((1, 1), jnp.float32),
            pltpu.VMEM((_BM, _BS), jnp.bfloat16),
            pltpu.VMEM((8, _N), jnp.bfloat16),
        ],
    )(a2, b2)
    return res[0, 0]


# speedup vs baseline: 1.1917x; 1.1917x over previous
import jax
import jax.numpy as jnp
from jax.experimental import pallas as pl
from jax.experimental.pallas import tpu as pltpu

_BM = 1024
_BN = 1024
_BS = 128
_N = 4096
_K = 64
_KP = 128


def _prep_body(x_ref, y_ref, a2_ref, b2_ref):
    x = x_ref[0]
    y = y_ref[0]
    xn = jnp.sum(x * x, axis=1, keepdims=True)
    yn = jnp.sum(y * y, axis=1, keepdims=True)
    ones = jnp.ones((_BM, 1), jnp.float32)
    zeros = jnp.zeros((_BM, _KP - _K - 2), jnp.float32)
    a2_ref[0] = jnp.concatenate(
        [x * -2.0, xn, ones, zeros], axis=1).astype(jnp.bfloat16)
    b2_ref[0] = jnp.concatenate(
        [y, ones, yn, zeros], axis=1).astype(jnp.bfloat16)


def _main_body(a2_ref, b2_ref, out_ref, acc_ref, racc_ref, cacc_ref):
    b_b = pl.program_id(0)
    b_i = pl.program_id(1)
    b_j = pl.program_id(2)
    nb = pl.num_programs(0)
    ni = pl.num_programs(1)
    nj = pl.num_programs(2)

    @pl.when((b_b == 0) & (b_i == 0) & (b_j == 0))
    def _init_acc():
        acc_ref[0, 0] = 0.0

    @pl.when(b_j == 0)
    def _init_racc():
        racc_ref[...] = jnp.full((_BM, _BS), jnp.inf, jnp.float32)

    @pl.when((b_i == 0) & (b_j == 0))
    def _init_cacc():
        cacc_ref[...] = jnp.full((8, _N), jnp.inf, jnp.float32)

    a2 = a2_ref[0]
    racc = racc_ref[...]
    for s in range(_BN // _BS):
        b2s = b2_ref[0, s * _BS:(s + 1) * _BS, :]
        d = jax.lax.dot_general(
            a2, b2s, (((1,), (1,)), ((), ())),
            preferred_element_type=jnp.float32)
        racc = jnp.minimum(racc, d)
        cp8 = jnp.min(d.reshape(_BM // 8, 8, _BS), axis=0)
        csl = (slice(None), pl.ds(b_j * _BN + s * _BS, _BS))
        cacc_ref[csl] = jnp.minimum(cacc_ref[csl], cp8)
    racc_ref[...] = racc

    @pl.when(b_j == nj - 1)
    def _fin_rows():
        acc_ref[0, 0] += jnp.sum(jnp.min(racc, axis=1))

    @pl.when(b_i == ni - 1)
    def _fin_cols():
        acc_ref[0, 0] += jnp.sum(
            jnp.min(cacc_ref[:, pl.ds(b_j * _BN, _BN)], axis=0))

    @pl.when((b_b == nb - 1) & (b_i == ni - 1) & (b_j == nj - 1))
    def _write_out():
        out_ref[...] = jnp.full((1, 1), acc_ref[0, 0], jnp.float32)


def kernel(input, output):
    nb, n, k = input.shape
    a2, b2 = pl.pallas_call(
        _prep_body,
        grid=(nb, n // _BM),
        in_specs=[
            pl.BlockSpec((1, _BM, k), lambda b, i: (b, i, 0)),
            pl.BlockSpec((1, _BM, k), lambda b, i: (b, i, 0)),
        ],
        out_specs=[
            pl.BlockSpec((1, _BM, _KP), lambda b, i: (b, i, 0)),
            pl.BlockSpec((1, _BM, _KP), lambda b, i: (b, i, 0)),
        ],
        out_shape=[
            jax.ShapeDtypeStruct((nb, n, _KP), jnp.bfloat16),
            jax.ShapeDtypeStruct((nb, n, _KP), jnp.bfloat16),
        ],
    )(input, output)

    res = pl.pallas_call(
        _main_body,
        grid=(nb, n // _BM, n // _BN),
        in_specs=[
            pl.BlockSpec((1, _BM, _KP), lambda b, i, j: (b, i, 0)),
            pl.BlockSpec((1, _BN, _KP), lambda b, i, j: (b, j, 0)),
        ],
        out_specs=pl.BlockSpec((1, 1), lambda b, i, j: (0, 0)),
        out_shape=jax.ShapeDtypeStruct((1, 1), jnp.float32),
        scratch_shapes=[
            pltpu.SMEM((1, 1), jnp.float32),
            pltpu.VMEM((_BM, _BS), jnp.float32),
            pltpu.VMEM((8, _N), jnp.float32),
        ],
    )(a2, b2)
    return res[0, 0]
